# baseline (speedup 1.0000x reference)
import functools

import jax
import jax.numpy as jnp
from jax import lax
from jax.experimental import pallas as pl
from jax.experimental.pallas import tpu as pltpu
from jax.experimental.pallas import tpu_sc as plsc

N_USERS = 1000000
N_ITEMS = 1000000
EMB = 32
LAYER = 64
N_GENRES = 32
GENRE_EMB = N_GENRES // 2
BATCH = 16384
PAD = 32

NC = 2
NS = 16
NW = NC * NS
B_PER_W = BATCH // NW
CHUNK = 64
N_CHUNKS = B_PER_W // CHUNK
NBUF = 2
N_TABLES = 4


N_SC_T = 2


def _sc_gather_body(iidx_hbm, t_imf, t_imlp, o_imf, o_imlp,
                    iidx_s, bufs, sems):
    wid = lax.axis_index("s") * NC + lax.axis_index("c")
    base = wid * B_PER_W

    pltpu.sync_copy(iidx_hbm.at[pl.ds(base, B_PER_W)], iidx_s)

    tables = (t_imf, t_imlp)
    outs = (o_imf, o_imlp)

    for c in range(N_CHUNKS):
        def fire(g, carry, c=c):
            j0 = c * CHUNK + g * 16
            vi = iidx_s[pl.ds(j0, 16)]
            for lane in range(16):
                for t in range(N_SC_T):
                    row = vi[lane]
                    pltpu.async_copy(
                        tables[t].at[pl.ds(row, 1)],
                        bufs.at[t, pl.ds(g * 16 + lane, 1)],
                        sems.at[t])
            return carry

        lax.fori_loop(0, CHUNK // 16, fire, 0)

        for t in range(N_SC_T):
            pltpu.make_async_copy(
                tables[t].at[pl.ds(0, CHUNK)], bufs.at[t], sems.at[t]
            ).wait()
        dst = pl.ds(base + c * CHUNK, CHUNK)
        for t in range(N_SC_T):
            pltpu.sync_copy(bufs.at[t], outs[t].at[dst])


_gathered_type = jax.ShapeDtypeStruct((BATCH, EMB), jnp.float32)


@functools.lru_cache(maxsize=1)
def _get_sc_gather():
    return pl.kernel(
        _sc_gather_body,
        out_type=[_gathered_type] * N_SC_T,
        mesh=plsc.VectorSubcoreMesh(core_axis_name="c", subcore_axis_name="s",
                                    num_cores=NC, num_subcores=NS),
        scratch_types=[
            pltpu.VMEM((B_PER_W,), jnp.int32),
            pltpu.VMEM((N_SC_T, CHUNK, EMB), jnp.float32),
            pltpu.SemaphoreType.DMA((N_SC_T,)),
        ],
    )


GBLK = 1024
N_GBLK = BATCH // GBLK


def _tc_gather_body(uidx_ref, t_umf, t_umlp, o_umf, o_umlp, sem):
    i = pl.program_id(0)
    base = i * GBLK

    def fire(j, carry):
        r = uidx_ref[j]
        dst = pl.ds(base + j, 1)
        pltpu.async_copy(t_umf.at[pl.ds(r, 1)], o_umf.at[dst], sem)
        pltpu.async_copy(t_umlp.at[pl.ds(r, 1)], o_umlp.at[dst], sem)
        return carry

    lax.fori_loop(0, GBLK, fire, 0)

    def drain(j, carry):
        pltpu.make_async_copy(t_umf.at[pl.ds(0, 1)], o_umf.at[pl.ds(0, 1)], sem).wait()
        pltpu.make_async_copy(t_umlp.at[pl.ds(0, 1)], o_umlp.at[pl.ds(0, 1)], sem).wait()
        return carry

    lax.fori_loop(0, GBLK, drain, 0)


def _tc_gather(uidx, t_umf, t_umlp):
    return pl.pallas_call(
        _tc_gather_body,
        grid=(N_GBLK,),
        in_specs=[
            pl.BlockSpec((GBLK,), lambda i: (i,), memory_space=pltpu.SMEM),
            pl.BlockSpec(memory_space=pl.ANY),
            pl.BlockSpec(memory_space=pl.ANY),
        ],
        out_specs=[pl.BlockSpec(memory_space=pl.ANY)] * 2,
        out_shape=[jax.ShapeDtypeStruct((BATCH, EMB), jnp.float32)] * 2,
        scratch_shapes=[pltpu.SemaphoreType.DMA],
    )(uidx, t_umf, t_umlp)


BLK = 2048
N_BLK = BATCH // BLK


def _tc_mlp_body(umf_ref, imf_ref, umlp_ref, imlp_ref, feats_ref, genre_ref,
                 w1_ref, b1_ref, w2_ref, b2_ref, w3_ref, b3_ref, out_ref):
    f32 = jnp.float32
    feats = feats_ref[0, 0, :]
    col = lax.broadcasted_iota(jnp.int32, (BLK, N_GENRES), 1)
    onehot = (feats[:, None] == col).astype(f32)
    ge = jnp.dot(onehot, genre_ref[...], preferred_element_type=f32)

    umlp = umlp_ref[:, 0:EMB]
    imlp = imlp_ref[:, 0:EMB]
    h1 = jnp.dot(umlp, w1_ref[0:EMB, :], preferred_element_type=f32)
    h1 += jnp.dot(imlp, w1_ref[EMB:2 * EMB, :], preferred_element_type=f32)
    h1 += jnp.dot(ge, w1_ref[2 * EMB:2 * EMB + GENRE_EMB, :], preferred_element_type=f32)
    h1 = jnp.maximum(h1 + b1_ref[...], 0.0)

    h2 = jnp.dot(h1, w2_ref[...], preferred_element_type=f32)
    h2 = jnp.maximum(h2 + b2_ref[...], 0.0)

    mf = umf_ref[:, 0:EMB] * imf_ref[:, 0:EMB]

    out = jnp.dot(h2, w3_ref[0:LAYER // 2, :], preferred_element_type=f32)
    out += jnp.dot(mf, w3_ref[LAYER // 2:LAYER, :], preferred_element_type=f32)
    out_ref[...] = out + b3_ref[0, 0]


def _tc_mlp(umf, imf, umlp, imlp, feats3, genre_emb, W1, b1, W2, b2, W3t, b3):
    emb_spec = pl.BlockSpec((BLK, PAD), lambda i: (i, 0))
    full = lambda shape: pl.BlockSpec(shape, lambda i: tuple(0 for _ in shape))
    return pl.pallas_call(
        _tc_mlp_body,
        grid=(N_BLK,),
        in_specs=[
            emb_spec, emb_spec, emb_spec, emb_spec,
            pl.BlockSpec((1, 1, BLK), lambda i: (i, 0, 0)),
            full((N_GENRES, GENRE_EMB)),
            full((2 * EMB + GENRE_EMB, LAYER)),
            full((1, LAYER)),
            full((LAYER, LAYER // 2)),
            full((1, LAYER // 2)),
            full((LAYER, 1)),
            full((1, 1)),
        ],
        out_specs=pl.BlockSpec((BLK, 1), lambda i: (i, 0)),
        out_shape=jax.ShapeDtypeStruct((BATCH, 1), jnp.float32),
    )(umf, imf, umlp, imlp, feats3, genre_emb, W1, b1, W2, b2, W3t, b3)


def kernel(user_indices, item_indices, feats, user_emb_mf, item_emb_mf,
           user_emb_mlp, item_emb_mlp, genre_emb, W1, b1, W2, b2, W3, b3):
    uidx = user_indices.astype(jnp.int32)
    iidx = item_indices.astype(jnp.int32)

    imf, imlp = _get_sc_gather()(iidx, item_emb_mf, item_emb_mlp)
    umf, umlp = _tc_gather(uidx, user_emb_mf, user_emb_mlp)

    feats3 = feats.astype(jnp.int32).reshape(N_BLK, 1, BLK)
    out2 = _tc_mlp(umf, imf, umlp, imlp, feats3, genre_emb,
                   W1, b1.reshape(1, LAYER), W2, b2.reshape(1, LAYER // 2),
                   W3, b3.reshape(1, 1))
    return out2.reshape(BATCH)


# speedup vs baseline: 1.4224x; 1.4224x over previous
import functools

import jax
import jax.numpy as jnp
from jax import lax
from jax.experimental import pallas as pl
from jax.experimental.pallas import tpu as pltpu
from jax.experimental.pallas import tpu_sc as plsc

N_USERS = 1000000
N_ITEMS = 1000000
EMB = 32
LAYER = 64
N_GENRES = 32
GENRE_EMB = N_GENRES // 2
BATCH = 16384
PAD = 32

NC = 2
NS = 16
NW = NC * NS
B_PER_W = BATCH // NW
CHUNK = 64
N_CHUNKS = B_PER_W // CHUNK
NBUF = 2
N_TABLES = 4


N_SC_T = 2


def _sc_gather_body(iidx_hbm, t_imf, t_imlp, o_imf, o_imlp,
                    iidx_s, bufs, sems):
    wid = lax.axis_index("s") * NC + lax.axis_index("c")
    base = wid * B_PER_W

    pltpu.sync_copy(iidx_hbm.at[pl.ds(base, B_PER_W)], iidx_s)

    tables = (t_imf, t_imlp)
    outs = (o_imf, o_imlp)

    for c in range(N_CHUNKS):
        def fire(g, carry, c=c):
            j0 = c * CHUNK + g * 16
            vi = iidx_s[pl.ds(j0, 16)]
            for lane in range(16):
                for t in range(N_SC_T):
                    row = vi[lane]
                    pltpu.async_copy(
                        tables[t].at[pl.ds(row, 1)],
                        bufs.at[t, pl.ds(g * 16 + lane, 1)],
                        sems.at[t])
            return carry

        lax.fori_loop(0, CHUNK // 16, fire, 0)

        for t in range(N_SC_T):
            pltpu.make_async_copy(
                tables[t].at[pl.ds(0, CHUNK)], bufs.at[t], sems.at[t]
            ).wait()
        dst = pl.ds(base + c * CHUNK, CHUNK)
        for t in range(N_SC_T):
            pltpu.sync_copy(bufs.at[t], outs[t].at[dst])


_gathered_type = jax.ShapeDtypeStruct((BATCH, EMB), jnp.float32)


@functools.lru_cache(maxsize=1)
def _get_sc_gather():
    return pl.kernel(
        _sc_gather_body,
        out_type=[_gathered_type] * N_SC_T,
        mesh=plsc.VectorSubcoreMesh(core_axis_name="c", subcore_axis_name="s",
                                    num_cores=NC, num_subcores=NS),
        scratch_types=[
            pltpu.VMEM((B_PER_W,), jnp.int32),
            pltpu.VMEM((N_SC_T, CHUNK, EMB), jnp.float32),
            pltpu.SemaphoreType.DMA((N_SC_T,)),
        ],
    )


GBLK = 1024
N_GBLK = BATCH // GBLK


def _tc_gather_body(uidx_ref, t_umf, t_umlp, o_umf, o_umlp, sem):
    i = pl.program_id(0)
    base = i * GBLK

    def fire(j, carry):
        r = uidx_ref[j]
        dst = pl.ds(base + j, 1)
        pltpu.async_copy(t_umf.at[pl.ds(r, 1)], o_umf.at[dst], sem)
        pltpu.async_copy(t_umlp.at[pl.ds(r, 1)], o_umlp.at[dst], sem)
        return carry

    lax.fori_loop(0, GBLK, fire, 0)

    def drain(j, carry):
        pltpu.make_async_copy(t_umf.at[pl.ds(0, 1)], o_umf.at[pl.ds(0, 1)], sem).wait()
        pltpu.make_async_copy(t_umlp.at[pl.ds(0, 1)], o_umlp.at[pl.ds(0, 1)], sem).wait()
        return carry

    lax.fori_loop(0, GBLK, drain, 0)


def _tc_gather(uidx, t_umf, t_umlp):
    return pl.pallas_call(
        _tc_gather_body,
        grid=(N_GBLK,),
        in_specs=[
            pl.BlockSpec((GBLK,), lambda i: (i,), memory_space=pltpu.SMEM),
            pl.BlockSpec(memory_space=pl.ANY),
            pl.BlockSpec(memory_space=pl.ANY),
        ],
        out_specs=[pl.BlockSpec(memory_space=pl.ANY)] * 2,
        out_shape=[jax.ShapeDtypeStruct((BATCH, EMB), jnp.float32)] * 2,
        scratch_shapes=[pltpu.SemaphoreType.DMA],
    )(uidx, t_umf, t_umlp)


BLK = 2048
N_BLK = BATCH // BLK


def _tc_mlp_body(umf_ref, imf_ref, umlp_ref, imlp_ref, feats_ref, genre_ref,
                 w1_ref, b1_ref, w2_ref, b2_ref, w3_ref, b3_ref, out_ref):
    f32 = jnp.float32
    feats = feats_ref[0, 0, :]
    col = lax.broadcasted_iota(jnp.int32, (BLK, N_GENRES), 1)
    onehot = (feats[:, None] == col).astype(f32)
    ge = jnp.dot(onehot, genre_ref[...], preferred_element_type=f32)

    umlp = umlp_ref[:, 0:EMB]
    imlp = imlp_ref[:, 0:EMB]
    h1 = jnp.dot(umlp, w1_ref[0:EMB, :], preferred_element_type=f32)
    h1 += jnp.dot(imlp, w1_ref[EMB:2 * EMB, :], preferred_element_type=f32)
    h1 += jnp.dot(ge, w1_ref[2 * EMB:2 * EMB + GENRE_EMB, :], preferred_element_type=f32)
    h1 = jnp.maximum(h1 + b1_ref[...], 0.0)

    h2 = jnp.dot(h1, w2_ref[...], preferred_element_type=f32)
    h2 = jnp.maximum(h2 + b2_ref[...], 0.0)

    mf = umf_ref[:, 0:EMB] * imf_ref[:, 0:EMB]

    out = jnp.dot(h2, w3_ref[0:LAYER // 2, :], preferred_element_type=f32)
    out += jnp.dot(mf, w3_ref[LAYER // 2:LAYER, :], preferred_element_type=f32)
    out_ref[...] = out + b3_ref[0, 0]


def _tc_mlp(umf, imf, umlp, imlp, feats3, genre_emb, W1, b1, W2, b2, W3t, b3):
    emb_spec = pl.BlockSpec((BLK, PAD), lambda i: (i, 0))
    full = lambda shape: pl.BlockSpec(shape, lambda i: tuple(0 for _ in shape))
    return pl.pallas_call(
        _tc_mlp_body,
        grid=(N_BLK,),
        in_specs=[
            emb_spec, emb_spec, emb_spec, emb_spec,
            pl.BlockSpec((1, 1, BLK), lambda i: (i, 0, 0)),
            full((N_GENRES, GENRE_EMB)),
            full((2 * EMB + GENRE_EMB, LAYER)),
            full((1, LAYER)),
            full((LAYER, LAYER // 2)),
            full((1, LAYER // 2)),
            full((LAYER, 1)),
            full((1, 1)),
        ],
        out_specs=pl.BlockSpec((BLK, 1), lambda i: (i, 0)),
        out_shape=jax.ShapeDtypeStruct((BATCH, 1), jnp.float32),
    )(umf, imf, umlp, imlp, feats3, genre_emb, W1, b1, W2, b2, W3t, b3)


def kernel(user_indices, item_indices, feats, user_emb_mf, item_emb_mf,
           user_emb_mlp, item_emb_mlp, genre_emb, W1, b1, W2, b2, W3, b3):
    uidx = user_indices.astype(jnp.int32)
    iidx = item_indices.astype(jnp.int32)

    imf, imlp = _get_sc_gather()(iidx, item_emb_mf, item_emb_mlp)
    umf, umlp = _get_sc_gather()(uidx, user_emb_mf, user_emb_mlp)

    feats3 = feats.astype(jnp.int32).reshape(N_BLK, 1, BLK)
    out2 = _tc_mlp(umf, imf, umlp, imlp, feats3, genre_emb,
                   W1, b1.reshape(1, LAYER), W2, b2.reshape(1, LAYER // 2),
                   W3, b3.reshape(1, 1))
    return out2.reshape(BATCH)
